# Initial kernel scaffold; baseline (speedup 1.0000x reference)
import jax
import jax.numpy as jnp
from jax.experimental import pallas as pl


def kernel(x_2d_in, unique_values, emb_table, gamma, beta):
    raise NotImplementedError("write your pallas kernel here")


# speedup vs baseline: 8.5346x; 8.5346x over previous
import functools

import jax
import jax.numpy as jnp
from jax.experimental import pallas as pl


def _histo_body(x_ref, uv_ref, cnt_ref):
    b = pl.program_id(0)
    fuel = x_ref[0, 0, :, :]
    uv3 = uv_ref[...][:, :, None]
    mask = (uv3 == fuel[None, :, :]).astype(jnp.float32)
    partial = jnp.sum(mask, axis=1)

    @pl.when(b == 0)
    def _init():
        cnt_ref[...] = partial

    @pl.when(b != 0)
    def _acc():
        cnt_ref[...] = cnt_ref[...] + partial


def _main_body(n_total, num_ch, x_ref, uv_ref, cnt_ref, embT_ref, g_ref,
               b_ref, o_ref):
    D = embT_ref.shape[0]
    fuel = x_ref[0, 3:4, :]
    mask = (uv_ref[...] == fuel).astype(jnp.float32)

    c_col = jnp.sum(cnt_ref[...], axis=1, keepdims=True)
    inv_n = 1.0 / float(n_total)
    embT = embT_ref[...]
    dims = (((1,), (0,)), ((), ()))
    mean = jax.lax.dot_general(embT, c_col, dims,
                               preferred_element_type=jnp.float32) * inv_n
    dev = embT - mean
    var = jax.lax.dot_general(dev * dev, c_col, dims,
                              preferred_element_type=jnp.float32) * inv_n
    scale = g_ref[...] * jax.lax.rsqrt(var + 1e-5)
    tnT = dev * scale + b_ref[...]

    bn = jax.lax.dot_general(tnT, mask, dims,
                             preferred_element_type=jnp.float32)
    o_ref[0, 0:3, :] = x_ref[0, 0:3, :]
    o_ref[0, 3:3 + D, :] = bn
    o_ref[0, 3 + D:, :] = x_ref[0, 4:num_ch, :]


def kernel(x_2d_in, unique_values, emb_table, gamma, beta):
    B, C, H, W = x_2d_in.shape
    K, D = emb_table.shape
    HW = H * W
    n_total = B * HW
    C_out = C - 1 + D

    x3 = x_2d_in.reshape(B, C, HW)
    uv_col = unique_values.reshape(K, 1)
    embT = emb_table.T
    g_col = gamma.reshape(D, 1)
    b_col = beta.reshape(D, 1)

    S = HW // 128
    counts = pl.pallas_call(
        _histo_body,
        grid=(B,),
        in_specs=[
            pl.BlockSpec((1, 1, S, 128), lambda b: (b, 3, 0, 0)),
            pl.BlockSpec((K, 1), lambda b: (0, 0)),
        ],
        out_specs=pl.BlockSpec((K, 128), lambda b: (0, 0)),
        out_shape=jax.ShapeDtypeStruct((K, 128), jnp.float32),
    )(x_2d_in.reshape(B, C, S, 128), uv_col)

    NS = 4
    L = HW // NS
    out3 = pl.pallas_call(
        functools.partial(_main_body, n_total, C),
        grid=(B, NS),
        in_specs=[
            pl.BlockSpec((1, C, L), lambda b, s: (b, 0, s)),
            pl.BlockSpec((K, 1), lambda b, s: (0, 0)),
            pl.BlockSpec((K, 128), lambda b, s: (0, 0)),
            pl.BlockSpec((D, K), lambda b, s: (0, 0)),
            pl.BlockSpec((D, 1), lambda b, s: (0, 0)),
            pl.BlockSpec((D, 1), lambda b, s: (0, 0)),
        ],
        out_specs=pl.BlockSpec((1, C_out, L), lambda b, s: (b, 0, s)),
        out_shape=jax.ShapeDtypeStruct((B, C_out, HW), jnp.float32),
    )(x3, uv_col, counts, embT, g_col, b_col)

    return out3.reshape(B, C_out, H, W)


# speedup vs baseline: 9.9048x; 1.1605x over previous
import functools

import jax
import jax.numpy as jnp
from jax import lax
from jax.experimental import pallas as pl
from jax.experimental.pallas import tpu as pltpu
from jax.experimental.pallas import tpu_sc as plsc


def _sc_histo_body(n_chunk, n_k, x_hbm, uv_hbm, out_hbm, buf_v, uvv_v,
                   acc_v):
    nc = 2
    wid = lax.axis_index("s") * nc + lax.axis_index("c")
    b = wid // 2
    half = wid % 2
    pltpu.sync_copy(x_hbm.at[b, 3, pl.ds(half * n_chunk, n_chunk)], buf_v)
    pltpu.sync_copy(uv_hbm, uvv_v)
    uvk = [uvv_v[k] for k in range(n_k)]
    zero = jnp.zeros((16,), jnp.float32)
    one = jnp.full((16,), 1.0, jnp.float32)

    def body(i, accs):
        v = buf_v[pl.ds(i * 16, 16)]
        return tuple(a + jnp.where(v == uvk[k], one, zero)
                     for k, a in enumerate(accs))

    accs = lax.fori_loop(0, n_chunk // 16, body, (zero,) * n_k)
    for k in range(n_k):
        acc_v[k] = accs[k]
    pltpu.sync_copy(acc_v, out_hbm.at[wid])


def _sc_histogram(x3, uv_bcast):
    B, C, HW = x3.shape
    K = uv_bcast.shape[0]
    NW = 32
    n_chunk = (B * HW) // NW
    mesh = plsc.VectorSubcoreMesh(core_axis_name="c", subcore_axis_name="s")
    return pl.kernel(
        functools.partial(_sc_histo_body, n_chunk, K),
        mesh=mesh,
        out_type=jax.ShapeDtypeStruct((NW, K, 16), jnp.float32),
        scratch_types=[
            pltpu.VMEM((n_chunk,), jnp.float32),
            pltpu.VMEM((K, 16), jnp.float32),
            pltpu.VMEM((K, 16), jnp.float32),
        ],
    )(x3, uv_bcast)


def _main_body(n_total, num_ch, x_ref, uv_ref, cnt_ref, embT_ref, g_ref,
               b_ref, o_ref):
    D = embT_ref.shape[0]
    fuel = x_ref[0, 3:4, :]
    mask = (uv_ref[...] == fuel).astype(jnp.float32)

    c_part = jnp.sum(cnt_ref[...], axis=0)
    c_col = jnp.sum(c_part, axis=1, keepdims=True)
    inv_n = 1.0 / float(n_total)
    embT = embT_ref[...]
    dims = (((1,), (0,)), ((), ()))
    mean = jax.lax.dot_general(embT, c_col, dims,
                               preferred_element_type=jnp.float32) * inv_n
    dev = embT - mean
    var = jax.lax.dot_general(dev * dev, c_col, dims,
                              preferred_element_type=jnp.float32) * inv_n
    scale = g_ref[...] * jax.lax.rsqrt(var + 1e-5)
    tnT = dev * scale + b_ref[...]

    bn = jax.lax.dot_general(tnT, mask, dims,
                             preferred_element_type=jnp.float32)
    o_ref[0, 0:3, :] = x_ref[0, 0:3, :]
    o_ref[0, 3:3 + D, :] = bn
    o_ref[0, 3 + D:, :] = x_ref[0, 4:num_ch, :]


def kernel(x_2d_in, unique_values, emb_table, gamma, beta):
    B, C, H, W = x_2d_in.shape
    K, D = emb_table.shape
    HW = H * W
    n_total = B * HW
    C_out = C - 1 + D

    x3 = x_2d_in.reshape(B, C, HW)
    uv_col = unique_values.reshape(K, 1)
    uv_bcast = jnp.broadcast_to(uv_col, (K, 16))
    embT = emb_table.T
    g_col = gamma.reshape(D, 1)
    b_col = beta.reshape(D, 1)

    counts = _sc_histogram(x3, uv_bcast)

    NS = 1
    L = HW // NS
    out3 = pl.pallas_call(
        functools.partial(_main_body, n_total, C),
        grid=(B, NS),
        in_specs=[
            pl.BlockSpec((1, C, L), lambda b, s: (b, 0, s)),
            pl.BlockSpec((K, 1), lambda b, s: (0, 0)),
            pl.BlockSpec((32, K, 16), lambda b, s: (0, 0, 0)),
            pl.BlockSpec((D, K), lambda b, s: (0, 0)),
            pl.BlockSpec((D, 1), lambda b, s: (0, 0)),
            pl.BlockSpec((D, 1), lambda b, s: (0, 0)),
        ],
        out_specs=pl.BlockSpec((1, C_out, L), lambda b, s: (b, 0, s)),
        out_shape=jax.ShapeDtypeStruct((B, C_out, HW), jnp.float32),
    )(x3, uv_col, counts, embT, g_col, b_col)

    return out3.reshape(B, C_out, H, W)
